# baseline (speedup 1.0000x reference)
import functools

import jax
import jax.numpy as jnp
from jax import lax
from jax.experimental import pallas as pl
from jax.experimental.pallas import tpu as pltpu
from jax.experimental.pallas import tpu_sc as plsc

SEQ_LEN = 200
BATCH = 4096
EMB_DIM = 64
VOCAB_ROWS = 2000000
HALF = EMB_DIM // 2
LANE = 128
B = SEQ_LEN * BATCH
NC, NS = 2, 16
NW = NC * NS
LPW = B // NW
CHUNK = 256
KG = CHUNK * 2 // LANE
NCHUNK = LPW // CHUNK
NBUF = 2
NOUT = NCHUNK // NBUF

_mesh = plsc.VectorSubcoreMesh(core_axis_name="c", subcore_axis_name="s")


@functools.partial(
    pl.kernel,
    mesh=_mesh,
    out_type=jax.ShapeDtypeStruct((B * 2, HALF), jnp.float32),
    scratch_types=[
        pltpu.VMEM((NBUF, KG, LANE), jnp.int32),
        pltpu.VMEM((NBUF, CHUNK * 2, HALF), jnp.float32),
        pltpu.SemaphoreType.DMA((NBUF,)),
        pltpu.SemaphoreType.DMA((NBUF,)),
    ],
    compiler_params=pltpu.CompilerParams(use_tc_tiling_on_sc=False),
)
def _emb_lookup(widx_hbm, table_hbm, out_hbm, idx_v, rows_v, gsem, wsem):
    wid = lax.axis_index("s") * NC + lax.axis_index("c")
    base = wid * LPW

    def body(t, carry):
        for b in range(NBUF):
            lk0 = base + (t * NBUF + b) * CHUNK

            @pl.when(t > 0)
            def _():
                pltpu.make_async_copy(
                    rows_v.at[b],
                    out_hbm.at[pl.ds((lk0 - NBUF * CHUNK) * 2, CHUNK * 2)],
                    wsem.at[b]).wait()

            for g in range(KG):
                pltpu.sync_copy(
                    widx_hbm.at[pl.ds((lk0 + g * (LANE // 2)) * 2, LANE)],
                    idx_v.at[b, g])
                pltpu.async_copy(
                    table_hbm.at[idx_v.at[b, g]],
                    rows_v.at[b, pl.ds(g * LANE, LANE)], gsem.at[b])
        for b in range(NBUF):
            lk0 = base + (t * NBUF + b) * CHUNK
            for g in range(KG):
                pltpu.make_async_copy(
                    table_hbm.at[idx_v.at[b, g]],
                    rows_v.at[b, pl.ds(g * LANE, LANE)], gsem.at[b]).wait()
            pltpu.async_copy(rows_v.at[b],
                             out_hbm.at[pl.ds(lk0 * 2, CHUNK * 2)],
                             wsem.at[b])
        return carry

    lax.fori_loop(0, NOUT, body, 0)
    for b in range(NBUF):
        lk0 = base + ((NOUT - 1) * NBUF + b) * CHUNK
        pltpu.make_async_copy(
            rows_v.at[b], out_hbm.at[pl.ds(lk0 * 2, CHUNK * 2)],
            wsem.at[b]).wait()


def kernel(words, lut_weight):
    w2 = words.astype(jnp.int32) * 2
    widx = jnp.stack([w2, w2 + 1], axis=-1).reshape(B * 2)
    out = _emb_lookup(widx, lut_weight.reshape(VOCAB_ROWS, HALF))
    return out.reshape(SEQ_LEN, BATCH, EMB_DIM)


# speedup vs baseline: 2.3046x; 2.3046x over previous
import functools

import jax
import jax.numpy as jnp
from jax import lax
from jax.experimental import pallas as pl
from jax.experimental.pallas import tpu as pltpu
from jax.experimental.pallas import tpu_sc as plsc

SEQ_LEN = 200
BATCH = 4096
EMB_DIM = 64
VOCAB = 1000000
WIDE = 128
LANE = 128
B = SEQ_LEN * BATCH
NC, NS = 2, 16
NW = NC * NS
LPW = B // NW
CHUNK = 256
KG = CHUNK // LANE
NCHUNK = LPW // CHUNK
NBUF = 2
NOUT = NCHUNK // NBUF

_mesh = plsc.VectorSubcoreMesh(core_axis_name="c", subcore_axis_name="s")


@functools.partial(
    pl.kernel,
    mesh=_mesh,
    out_type=jax.ShapeDtypeStruct((SEQ_LEN, BATCH, WIDE), jnp.float32),
    scratch_types=[
        pltpu.VMEM((NBUF, KG, LANE), jnp.int32),
        pltpu.VMEM((NBUF, CHUNK, WIDE), jnp.float32),
        pltpu.SemaphoreType.DMA((NBUF,)),
        pltpu.SemaphoreType.DMA((NBUF,)),
    ],
)
def _emb_lookup(words_hbm, table_hbm, out_hbm, idx_v, rows_v, gsem, wsem):
    wid = lax.axis_index("s") * NC + lax.axis_index("c")
    base = wid * LPW

    def body(t, carry):
        for b in range(NBUF):
            lk0 = base + (t * NBUF + b) * CHUNK

            @pl.when(t > 0)
            def _():
                pv = lk0 - NBUF * CHUNK
                pltpu.make_async_copy(
                    rows_v.at[b],
                    out_hbm.at[pv // BATCH, pl.ds(pv % BATCH, CHUNK)],
                    wsem.at[b]).wait()

            for g in range(KG):
                lk = lk0 + g * LANE
                pltpu.sync_copy(
                    words_hbm.at[lk // BATCH, pl.ds(lk % BATCH, LANE)],
                    idx_v.at[b, g])
                pltpu.async_copy(
                    table_hbm.at[idx_v.at[b, g]],
                    rows_v.at[b, pl.ds(g * LANE, LANE)], gsem.at[b])
        for b in range(NBUF):
            lk0 = base + (t * NBUF + b) * CHUNK
            for g in range(KG):
                pltpu.make_async_copy(
                    table_hbm.at[idx_v.at[b, g]],
                    rows_v.at[b, pl.ds(g * LANE, LANE)], gsem.at[b]).wait()
            pltpu.async_copy(
                rows_v.at[b],
                out_hbm.at[lk0 // BATCH, pl.ds(lk0 % BATCH, CHUNK)],
                wsem.at[b])
        return carry

    lax.fori_loop(0, NOUT, body, 0)
    for b in range(NBUF):
        lk0 = base + ((NOUT - 1) * NBUF + b) * CHUNK
        pltpu.make_async_copy(
            rows_v.at[b],
            out_hbm.at[lk0 // BATCH, pl.ds(lk0 % BATCH, CHUNK)],
            wsem.at[b]).wait()


def kernel(words, lut_weight):
    table_wide = jnp.pad(lut_weight, ((0, 0), (0, WIDE - EMB_DIM)))
    out = _emb_lookup(words.astype(jnp.int32), table_wide)
    return out[..., :EMB_DIM]
